# Initial kernel scaffold; baseline (speedup 1.0000x reference)
import jax
import jax.numpy as jnp
from jax.experimental import pallas as pl


def kernel(x1, x2, edge_index1, edge_index2, E0, E1t, E2t, W1, as1, ad1, b1, W2, as2, ad2, b2, smoothing, Wk, A):
    raise NotImplementedError("write your pallas kernel here")


# speedup vs baseline: 8.3896x; 8.3896x over previous
import functools

import jax
import jax.numpy as jnp
from jax.experimental import pallas as pl

PAST = 12
FUTURE = 12
OUTP = 31
HEADS = 4
NEG = -jnp.inf
HI = jax.lax.Precision.HIGHEST


def _onehot_matmul(idx, table):
    rows = table.shape[0]
    oh = (idx[:, None] == jax.lax.broadcasted_iota(jnp.int32, (idx.shape[0], rows), 1)
          ).astype(jnp.float32)
    return jnp.dot(oh, table, precision=HI)


def _hist(sl, dl, gb, epg, span):
    bins = sl * span + dl
    nbin = span * span
    oh = (bins[:, :, None] == jax.lax.broadcasted_iota(
        jnp.int32, (gb, epg, nbin), 2)).astype(jnp.float32)
    c = oh.sum(axis=1).reshape(gb, span, span)
    eye = (jax.lax.broadcasted_iota(jnp.int32, (span, span), 0)
           == jax.lax.broadcasted_iota(jnp.int32, (span, span), 1)
           ).astype(jnp.float32)
    return c + eye[None]


def _gat_dense(xfeat, cmat, w, a_s, a_d, bias, gb, oc):
    xp = jnp.dot(xfeat, w).reshape(gb, PAST, HEADS, oc)
    asr = (xp * a_s[None, None]).sum(-1)
    adt = (xp * a_d[None, None]).sum(-1)
    al = asr[:, :, None, :] + adt[:, None, :, :]
    al = jnp.where(al >= 0, al, 0.2 * al)
    maskneg = (jnp.minimum(cmat, 1.0) - 1.0) * 1e30
    amax = jnp.max(al + maskneg[:, :, :, None], axis=1, keepdims=True)
    ex = jnp.exp(al - amax) * cmat[:, :, :, None]
    den = ex.sum(axis=1, keepdims=True)
    att = ex / (den + 1e-16)
    acc = jnp.zeros((gb, PAST, oc), jnp.float32)
    for h in range(HEADS):
        acc = acc + (att[:, :, :, h, None] * xp[:, :, None, h, :]).sum(axis=1)
    return acc * (1.0 / HEADS) + bias[None]


def _body(x1_ref, x2_ref, s1_ref, d1_ref, s2_ref, d2_ref, e0_ref, e1t_ref,
          e2t_ref, w1_ref, as1_ref, ad1_ref, b1_ref, w2_ref, as2_ref, ad2_ref,
          b2_ref, sm_ref, wk_ref, a_ref, o_ref, *, gb, epg):
    def preprocess(xb):
        i0 = xb[:, 0].astype(jnp.int32)
        i1 = xb[:, 1].astype(jnp.int32)
        i2 = xb[:, 2].astype(jnp.int32)
        return jnp.concatenate(
            [_onehot_matmul(i0, e0_ref[...]),
             _onehot_matmul(i1, e1t_ref[...]),
             _onehot_matmul(i2, e2t_ref[...]),
             xb[:, 3:]], axis=-1)

    x1b = x1_ref[...]
    x = preprocess(x1b)
    x2p = preprocess(x2_ref[...]).reshape(gb, FUTURE, OUTP)

    c1 = _hist(s1_ref[...] % PAST, d1_ref[...] % PAST, gb, epg, PAST)
    c2 = _hist(s2_ref[...] % FUTURE, d2_ref[...] % FUTURE, gb, epg, FUTURE)

    h1 = _gat_dense(x, c1, w1_ref[...], as1_ref[...], ad1_ref[...],
                    b1_ref[...], gb, OUTP - 1)

    xk_in = jnp.concatenate([h1, x2p[:, :, :OUTP - 1]], axis=1)
    nrows = PAST + FUTURE
    mu = xk_in.mean(axis=1, keepdims=True)
    var = ((xk_in - mu) ** 2).sum(axis=1, keepdims=True) / (nrows - 1)
    xk = (xk_in - mu) / jnp.sqrt(var)
    x_p = xk[:, :PAST]
    x_f = xk[:, PAST:]

    sm = jax.nn.sigmoid(sm_ref[0, 0])
    diff = (x_p[:, None, :, :] - x_f[:, :, None, :]) / (sm * 0.01)
    s0 = jnp.dot(wk_ref[...], wk_ref[...].T)
    s_mat = s0 + s0.T
    d2 = jnp.dot(diff.reshape(gb * PAST * FUTURE, OUTP - 1), s_mat
                 ).reshape(gb, FUTURE, PAST, OUTP - 1)
    d2b = d2.astype(jnp.bfloat16).astype(jnp.float32)
    dfb = diff.astype(jnp.bfloat16).astype(jnp.float32)
    q = (d2b * dfb).sum(-1)
    al2 = -0.5 * q
    a_tmp = a_ref[...][:PAST, PAST:].T
    a_m = ((a_tmp != 0.0).astype(jnp.float32) - 1.0) * 1e30
    al2 = al2 + a_m[None]
    m = al2.max(axis=-1, keepdims=True)
    e = jnp.exp(al2 - m)
    p_att = e / e.sum(axis=-1, keepdims=True)

    y = x1b[:, 5].reshape(gb, PAST)
    pb = p_att.astype(jnp.bfloat16).astype(jnp.float32)
    yb = y.astype(jnp.bfloat16).astype(jnp.float32)
    yh = (pb * yb[:, None, :]).sum(-1)

    x2c = jnp.concatenate([x2p[:, :, :OUTP - 1], yh[:, :, None]],
                          axis=-1).reshape(gb * FUTURE, OUTP)
    out2 = _gat_dense(x2c, c2, w2_ref[...], as2_ref[...], ad2_ref[...],
                      b2_ref[...], gb, 1)
    o_ref[...] = out2[:, :, 0]


@jax.jit
def kernel(x1, x2, edge_index1, edge_index2, E0, E1t, E2t, W1, as1, ad1, b1,
           W2, as2, ad2, b2, smoothing, Wk, A):
    g = x1.shape[0] // PAST
    epg = edge_index1.shape[1] // g
    gb = 8 if g % 8 == 0 else g
    nblk = g // gb

    s1 = edge_index1[0].reshape(g, epg)
    d1 = edge_index1[1].reshape(g, epg)
    s2 = edge_index2[0].reshape(g, epg)
    d2 = edge_index2[1].reshape(g, epg)
    b1_2 = b1.reshape(1, -1)
    b2_2 = b2.reshape(1, -1)
    sm2 = smoothing.reshape(1, 1)

    def full(a):
        return pl.BlockSpec(a.shape, lambda b: tuple(0 for _ in a.shape))

    in_specs = [
        pl.BlockSpec((gb * PAST, 6), lambda b: (b, 0)),
        pl.BlockSpec((gb * FUTURE, 6), lambda b: (b, 0)),
        pl.BlockSpec((gb, epg), lambda b: (b, 0)),
        pl.BlockSpec((gb, epg), lambda b: (b, 0)),
        pl.BlockSpec((gb, epg), lambda b: (b, 0)),
        pl.BlockSpec((gb, epg), lambda b: (b, 0)),
        full(E0), full(E1t), full(E2t), full(W1), full(as1), full(ad1),
        full(b1_2), full(W2), full(as2), full(ad2), full(b2_2), full(sm2),
        full(Wk), full(A),
    ]
    out = pl.pallas_call(
        functools.partial(_body, gb=gb, epg=epg),
        grid=(nblk,),
        in_specs=in_specs,
        out_specs=pl.BlockSpec((gb, FUTURE), lambda b: (b, 0)),
        out_shape=jax.ShapeDtypeStruct((g, FUTURE), jnp.float32),
    )(x1, x2, s1, d1, s2, d2, E0, E1t, E2t, W1, as1, ad1, b1_2,
      W2, as2, ad2, b2_2, sm2, Wk, A)
    return out


# speedup vs baseline: 32.7697x; 3.9060x over previous
import functools

import jax
import jax.numpy as jnp
from jax.experimental import pallas as pl

PAST = 12
FUTURE = 12
OUTP = 31
HEADS = 4
HI = jax.lax.Precision.HIGHEST


def _onehot_matmul(idx, table):
    rows = table.shape[0]
    oh = (idx[:, None] == jax.lax.broadcasted_iota(jnp.int32, (idx.shape[0], rows), 1)
          ).astype(jnp.float32)
    return jnp.dot(oh, table, precision=HI)


def _hist(sl, dl, gb, epg, span):
    bins = sl * span + dl
    nbin = span * span
    oh = (bins[:, :, None] == jax.lax.broadcasted_iota(
        jnp.int32, (gb, epg, nbin), 2)).astype(jnp.float32)
    c = oh.sum(axis=1).reshape(gb, span, span)
    eye = (jax.lax.broadcasted_iota(jnp.int32, (span, span), 0)
           == jax.lax.broadcasted_iota(jnp.int32, (span, span), 1)
           ).astype(jnp.float32)
    return c + eye[None]


def _bdot(a, b):
    return jax.lax.dot_general(a, b, (((2,), (1,)), ((0,), (0,))),
                               precision=HI)


def _gat_dense(xfeat, cmat, w, a_s, a_d, bias, gb, oc):
    xp2d = jnp.dot(xfeat, w)
    maskneg = (jnp.minimum(cmat, 1.0) - 1.0) * 1e30
    acc = jnp.zeros((gb, PAST, oc), jnp.float32)
    for h in range(HEADS):
        xph = xp2d[:, h * oc:(h + 1) * oc]
        asr = (xph * a_s[h][None, :]).sum(-1).reshape(gb, PAST)
        adt = (xph * a_d[h][None, :]).sum(-1).reshape(gb, PAST)
        al = asr[:, :, None] + adt[:, None, :]
        al = jnp.where(al >= 0, al, 0.2 * al)
        amax = jnp.max(al + maskneg, axis=1, keepdims=True)
        ex = jnp.exp(al - amax) * cmat
        den = ex.sum(axis=1, keepdims=True)
        att = ex / (den + 1e-16)
        acc = acc + _bdot(att.transpose(0, 2, 1), xph.reshape(gb, PAST, oc))
    return acc * (1.0 / HEADS) + bias[None]


def _body(x1_ref, x2_ref, s1_ref, d1_ref, s2_ref, d2_ref, e0_ref, e1t_ref,
          e2t_ref, w1_ref, as1_ref, ad1_ref, b1_ref, w2_ref, as2_ref, ad2_ref,
          b2_ref, sm_ref, wk_ref, a_ref, o_ref, *, gb, epg):
    def preprocess(xb):
        i0 = xb[:, 0].astype(jnp.int32)
        i1 = xb[:, 1].astype(jnp.int32)
        i2 = xb[:, 2].astype(jnp.int32)
        return jnp.concatenate(
            [_onehot_matmul(i0, e0_ref[...]),
             _onehot_matmul(i1, e1t_ref[...]),
             _onehot_matmul(i2, e2t_ref[...]),
             xb[:, 3:]], axis=-1)

    x1b = x1_ref[...]
    x = preprocess(x1b)
    x2p = preprocess(x2_ref[...])
    x2pk = x2p[:, :OUTP - 1].reshape(gb, FUTURE, OUTP - 1)

    c1 = _hist(s1_ref[...] % PAST, d1_ref[...] % PAST, gb, epg, PAST)
    c2 = _hist(s2_ref[...] % FUTURE, d2_ref[...] % FUTURE, gb, epg, FUTURE)

    h1 = _gat_dense(x, c1, w1_ref[...], as1_ref[...], ad1_ref[...],
                    b1_ref[...], gb, OUTP - 1)

    xk_in = jnp.concatenate([h1, x2pk], axis=1)
    nrows = PAST + FUTURE
    mu = xk_in.mean(axis=1, keepdims=True)
    var = ((xk_in - mu) ** 2).sum(axis=1, keepdims=True) / (nrows - 1)
    xk = (xk_in - mu) / jnp.sqrt(var)
    x_p = xk[:, :PAST]
    x_f = xk[:, PAST:]

    sm = jax.nn.sigmoid(sm_ref[0, 0])
    xpr = jnp.tile(x_p, (1, FUTURE, 1))
    xfr = jnp.repeat(x_f, PAST, axis=1)
    diff = ((xpr - xfr) / (sm * 0.01)).reshape(gb * PAST * FUTURE, OUTP - 1)
    s0 = jnp.dot(wk_ref[...], wk_ref[...].T)
    s_mat = s0 + s0.T
    d2 = jnp.dot(diff, s_mat)
    d2b = d2.astype(jnp.bfloat16).astype(jnp.float32)
    dfb = diff.astype(jnp.bfloat16).astype(jnp.float32)
    q = (d2b * dfb).sum(-1).reshape(gb, FUTURE, PAST)
    al2 = -0.5 * q
    a_tmp = a_ref[...][:PAST, PAST:].T
    a_m = ((a_tmp != 0.0).astype(jnp.float32) - 1.0) * 1e30
    al2 = al2 + a_m[None]
    m = al2.max(axis=-1, keepdims=True)
    e = jnp.exp(al2 - m)
    p_att = e / e.sum(axis=-1, keepdims=True)

    y = x1b[:, 5].reshape(gb, PAST)
    pb = p_att.astype(jnp.bfloat16).astype(jnp.float32)
    yb = y.astype(jnp.bfloat16).astype(jnp.float32)
    yh = (pb * yb[:, None, :]).sum(-1)

    x2c = jnp.concatenate(
        [x2p[:, :OUTP - 1], yh.reshape(gb * FUTURE, 1)], axis=-1)
    out2 = _gat_dense(x2c, c2, w2_ref[...], as2_ref[...], ad2_ref[...],
                      b2_ref[...], gb, 1)
    o_ref[...] = out2[:, :, 0]


@jax.jit
def kernel(x1, x2, edge_index1, edge_index2, E0, E1t, E2t, W1, as1, ad1, b1,
           W2, as2, ad2, b2, smoothing, Wk, A):
    g = x1.shape[0] // PAST
    epg = edge_index1.shape[1] // g
    gb = 8 if g % 8 == 0 else g
    nblk = g // gb

    s1 = edge_index1[0].reshape(g, epg)
    d1 = edge_index1[1].reshape(g, epg)
    s2 = edge_index2[0].reshape(g, epg)
    d2 = edge_index2[1].reshape(g, epg)
    b1_2 = b1.reshape(1, -1)
    b2_2 = b2.reshape(1, -1)
    sm2 = smoothing.reshape(1, 1)

    def full(a):
        return pl.BlockSpec(a.shape, lambda b: tuple(0 for _ in a.shape))

    in_specs = [
        pl.BlockSpec((gb * PAST, 6), lambda b: (b, 0)),
        pl.BlockSpec((gb * FUTURE, 6), lambda b: (b, 0)),
        pl.BlockSpec((gb, epg), lambda b: (b, 0)),
        pl.BlockSpec((gb, epg), lambda b: (b, 0)),
        pl.BlockSpec((gb, epg), lambda b: (b, 0)),
        pl.BlockSpec((gb, epg), lambda b: (b, 0)),
        full(E0), full(E1t), full(E2t), full(W1), full(as1), full(ad1),
        full(b1_2), full(W2), full(as2), full(ad2), full(b2_2), full(sm2),
        full(Wk), full(A),
    ]
    out = pl.pallas_call(
        functools.partial(_body, gb=gb, epg=epg),
        grid=(nblk,),
        in_specs=in_specs,
        out_specs=pl.BlockSpec((gb, FUTURE), lambda b: (b, 0)),
        out_shape=jax.ShapeDtypeStruct((g, FUTURE), jnp.float32),
    )(x1, x2, s1, d1, s2, d2, E0, E1t, E2t, W1, as1, ad1, b1_2,
      W2, as2, ad2, b2_2, sm2, Wk, A)
    return out


# speedup vs baseline: 43.5490x; 1.3289x over previous
import functools

import jax
import jax.numpy as jnp
from jax.experimental import pallas as pl

PAST = 12
FUTURE = 12
OUTP = 31
HEADS = 4
HI = jax.lax.Precision.HIGHEST


def _onehot_matmul(idx, table):
    rows = table.shape[0]
    oh = (idx[:, None] == jax.lax.broadcasted_iota(jnp.int32, (idx.shape[0], rows), 1)
          ).astype(jnp.float32)
    return jnp.dot(oh, table, precision=HI)


def _hist(sl, dl, gb, epg, span):
    bins = sl * span + dl
    nbin = span * span
    oh = (bins[:, :, None] == jax.lax.broadcasted_iota(
        jnp.int32, (gb, epg, nbin), 2)).astype(jnp.float32)
    c = oh.sum(axis=1).reshape(gb, span, span)
    eye = (jax.lax.broadcasted_iota(jnp.int32, (span, span), 0)
           == jax.lax.broadcasted_iota(jnp.int32, (span, span), 1)
           ).astype(jnp.float32)
    return c + eye[None]


def _bdot(a, b):
    return jax.lax.dot_general(a, b, (((2,), (1,)), ((0,), (0,))),
                               precision=HI)


def _gat_dense(xfeat, cmat, w, a_s, a_d, bias, gb, oc):
    xp2d = jnp.dot(xfeat, w)
    maskneg = (jnp.minimum(cmat, 1.0) - 1.0) * 1e30
    acc = jnp.zeros((gb, PAST, oc), jnp.float32)
    for h in range(HEADS):
        xph = xp2d[:, h * oc:(h + 1) * oc]
        asr = (xph * a_s[h][None, :]).sum(-1).reshape(gb, PAST)
        adt = (xph * a_d[h][None, :]).sum(-1).reshape(gb, PAST)
        al = asr[:, :, None] + adt[:, None, :]
        al = jnp.where(al >= 0, al, 0.2 * al)
        amax = jnp.max(al + maskneg, axis=1, keepdims=True)
        ex = jnp.exp(al - amax) * cmat
        den = ex.sum(axis=1, keepdims=True)
        att = ex / (den + 1e-16)
        acc = acc + _bdot(att.transpose(0, 2, 1), xph.reshape(gb, PAST, oc))
    return acc * (1.0 / HEADS) + bias[None]


def _body(x1_ref, x2_ref, s1_ref, d1_ref, s2_ref, d2_ref, e0_ref, e1t_ref,
          e2t_ref, w1_ref, as1_ref, ad1_ref, b1_ref, w2_ref, as2_ref, ad2_ref,
          b2_ref, sm_ref, wk_ref, a_ref, o_ref, *, gb, epg):
    def preprocess(xb):
        i0 = xb[:, 0].astype(jnp.int32)
        i1 = xb[:, 1].astype(jnp.int32)
        i2 = xb[:, 2].astype(jnp.int32)
        return jnp.concatenate(
            [_onehot_matmul(i0, e0_ref[...]),
             _onehot_matmul(i1, e1t_ref[...]),
             _onehot_matmul(i2, e2t_ref[...]),
             xb[:, 3:]], axis=-1)

    x1b = x1_ref[...]
    x = preprocess(x1b)
    x2p = preprocess(x2_ref[...])
    x2pk = x2p[:, :OUTP - 1].reshape(gb, FUTURE, OUTP - 1)

    c1 = _hist(s1_ref[...] % PAST, d1_ref[...] % PAST, gb, epg, PAST)
    c2 = _hist(s2_ref[...] % FUTURE, d2_ref[...] % FUTURE, gb, epg, FUTURE)

    h1 = _gat_dense(x, c1, w1_ref[...], as1_ref[...], ad1_ref[...],
                    b1_ref[...], gb, OUTP - 1)

    xk_in = jnp.concatenate([h1, x2pk], axis=1)
    nrows = PAST + FUTURE
    mu = xk_in.mean(axis=1, keepdims=True)
    var = ((xk_in - mu) ** 2).sum(axis=1, keepdims=True) / (nrows - 1)
    xk = (xk_in - mu) / jnp.sqrt(var)
    x_p = xk[:, :PAST]
    x_f = xk[:, PAST:]

    sm = jax.nn.sigmoid(sm_ref[0, 0])
    xpr = jnp.tile(x_p, (1, FUTURE, 1))
    xfr = jnp.repeat(x_f, PAST, axis=1)
    diff = ((xpr - xfr) / (sm * 0.01)).reshape(gb * PAST * FUTURE, OUTP - 1)
    s0 = jnp.dot(wk_ref[...], wk_ref[...].T)
    s_mat = s0 + s0.T
    d2 = jnp.dot(diff, s_mat)
    d2b = d2.astype(jnp.bfloat16).astype(jnp.float32)
    dfb = diff.astype(jnp.bfloat16).astype(jnp.float32)
    q = (d2b * dfb).sum(-1).reshape(gb, FUTURE, PAST)
    al2 = -0.5 * q
    a_tmp = a_ref[...][:PAST, PAST:].T
    a_m = ((a_tmp != 0.0).astype(jnp.float32) - 1.0) * 1e30
    al2 = al2 + a_m[None]
    m = al2.max(axis=-1, keepdims=True)
    e = jnp.exp(al2 - m)
    p_att = e / e.sum(axis=-1, keepdims=True)

    y = x1b[:, 5].reshape(gb, PAST)
    pb = p_att.astype(jnp.bfloat16).astype(jnp.float32)
    yb = y.astype(jnp.bfloat16).astype(jnp.float32)
    yh = (pb * yb[:, None, :]).sum(-1)

    x2c = jnp.concatenate(
        [x2p[:, :OUTP - 1], yh.reshape(gb * FUTURE, 1)], axis=-1)
    out2 = _gat_dense(x2c, c2, w2_ref[...], as2_ref[...], ad2_ref[...],
                      b2_ref[...], gb, 1)
    o_ref[...] = out2[:, :, 0]


@jax.jit
def kernel(x1, x2, edge_index1, edge_index2, E0, E1t, E2t, W1, as1, ad1, b1,
           W2, as2, ad2, b2, smoothing, Wk, A):
    g = x1.shape[0] // PAST
    epg = edge_index1.shape[1] // g
    gb = 32 if g % 32 == 0 else g
    nblk = g // gb

    s1 = edge_index1[0].reshape(g, epg)
    d1 = edge_index1[1].reshape(g, epg)
    s2 = edge_index2[0].reshape(g, epg)
    d2 = edge_index2[1].reshape(g, epg)
    b1_2 = b1.reshape(1, -1)
    b2_2 = b2.reshape(1, -1)
    sm2 = smoothing.reshape(1, 1)

    def full(a):
        return pl.BlockSpec(a.shape, lambda b: tuple(0 for _ in a.shape))

    in_specs = [
        pl.BlockSpec((gb * PAST, 6), lambda b: (b, 0)),
        pl.BlockSpec((gb * FUTURE, 6), lambda b: (b, 0)),
        pl.BlockSpec((gb, epg), lambda b: (b, 0)),
        pl.BlockSpec((gb, epg), lambda b: (b, 0)),
        pl.BlockSpec((gb, epg), lambda b: (b, 0)),
        pl.BlockSpec((gb, epg), lambda b: (b, 0)),
        full(E0), full(E1t), full(E2t), full(W1), full(as1), full(ad1),
        full(b1_2), full(W2), full(as2), full(ad2), full(b2_2), full(sm2),
        full(Wk), full(A),
    ]
    out = pl.pallas_call(
        functools.partial(_body, gb=gb, epg=epg),
        grid=(nblk,),
        in_specs=in_specs,
        out_specs=pl.BlockSpec((gb, FUTURE), lambda b: (b, 0)),
        out_shape=jax.ShapeDtypeStruct((g, FUTURE), jnp.float32),
    )(x1, x2, s1, d1, s2, d2, E0, E1t, E2t, W1, as1, ad1, b1_2,
      W2, as2, ad2, b2_2, sm2, Wk, A)
    return out
